# baseline (speedup 1.0000x reference)
import functools

import jax
import jax.numpy as jnp
from jax import lax
from jax.experimental import pallas as pl
from jax.experimental.pallas import tpu as pltpu
from jax.experimental.pallas import tpu_sc as plsc

_NW = 32
_G = 16
_N = 2048


def _drain(v):
    s = v[0]
    for l in range(1, _G):
        s = s + v[l]
    return s


def _sc_body(pred_hbm, target_hbm, outa_hbm, outr_hbm,
             p_buf, t_buf, a_row, r_row, outa_v, outr_v, sem0, sem1):
    wid = lax.axis_index("s") * 2 + lax.axis_index("c")
    rows_per_w = outa_v.shape[0]
    lane = lax.iota(jnp.int32, _G)
    g_base = wid * rows_per_w
    nchunk = _N // _G

    def start_dma(row, slot_p, slot_t, s):
        g = g_base + row
        cp = pltpu.make_async_copy(pred_hbm.at[g // _N, g % _N], slot_p, s)
        ct = pltpu.make_async_copy(target_hbm.at[g // _N, g % _N], slot_t, s)
        cp.start()
        ct.start()
        return cp, ct

    start_dma(0, p_buf.at[0], t_buf.at[0], sem0)
    start_dma(1, p_buf.at[1], t_buf.at[1], sem1)

    def row_work(row, slot, s):
        g = g_base + row
        pltpu.make_async_copy(pred_hbm.at[g // _N, g % _N], p_buf.at[slot], s).wait()
        pltpu.make_async_copy(target_hbm.at[g // _N, g % _N], t_buf.at[slot], s).wait()
        i_col = (g_base + row) % _N
        p_r = p_buf.at[slot]
        t_r = t_buf.at[slot]

        def prep_c(c, acc):
            cA, cR = acc
            col = c * _G + lane
            p16 = p_r[pl.ds(c * _G, _G)]
            t16 = t_r[pl.ds(c * _G, _G)]
            vt = jnp.where(col == i_col, 0.0, t16)
            nt = 1.0 - t16
            a_row[pl.ds(c * _G, _G)] = p16 * vt
            r_row[pl.ds(c * _G, _G)] = (1.0 - p16) * nt
            return (cA + vt, cR + nt)

        zf = jnp.zeros((_G,), jnp.float32)
        cAv, cRv = lax.fori_loop(0, nchunk, prep_c, (zf, zf), unroll=4)
        kA1 = (_drain(cAv) * 0.3).astype(jnp.int32) + 1
        kR1 = (_drain(cRv) * 0.3).astype(jnp.int32) + 1

        loA = jnp.float32(0.0)
        hiA = jnp.float32(1.001)
        loR = jnp.float32(0.0)
        hiR = jnp.float32(1.001)
        zi = jnp.zeros((_G,), jnp.int32)
        one = jnp.ones((_G,), jnp.int32)
        for _ in range(18):
            midA = (loA + hiA) * 0.5
            midR = (loR + hiR) * 0.5

            def cnt_c(c, acc, _mA=midA, _mR=midR):
                aA, aR = acc
                a16 = a_row[pl.ds(c * _G, _G)]
                r16 = r_row[pl.ds(c * _G, _G)]
                aA = aA + jnp.where(a16 >= _mA, one, zi)
                aR = aR + jnp.where(r16 >= _mR, one, zi)
                return (aA, aR)

            accA, accR = lax.fori_loop(0, nchunk, cnt_c, (zi, zi), unroll=8)
            okA = _drain(accA) >= kA1
            okR = _drain(accR) >= kR1
            loA = jnp.where(okA, midA, loA)
            hiA = jnp.where(okA, hiA, midA)
            loR = jnp.where(okR, midR, loR)
            hiR = jnp.where(okR, hiR, midR)

        def sum_c(c, acc):
            sA, mAc, sR, mRc = acc
            col = c * _G + lane
            a16 = a_row[pl.ds(c * _G, _G)]
            r16 = r_row[pl.ds(c * _G, _G)]
            t16 = t_r[pl.ds(c * _G, _G)]
            vt = jnp.where(col == i_col, 0.0, t16)
            nt = 1.0 - t16
            mA = a16 <= hiA
            mR = r16 <= hiR
            return (sA + jnp.where(mA, a16, 0.0),
                    mAc + jnp.where(mA, vt, 0.0),
                    sR + jnp.where(mR, r16, 0.0),
                    mRc + jnp.where(mR, nt, 0.0))

        sA, mAc, sR, mRc = lax.fori_loop(
            0, nchunk, sum_c, (zf, zf, zf, zf), unroll=4)
        den_a = _drain(mAc)
        den_r = _drain(mRc)
        zv = jnp.zeros((_G,), jnp.float32)
        apa = (zv + _drain(sA)) / (zv + jnp.where(den_a > 0, den_a, 1.0))
        apr = (zv + _drain(sR)) / (zv + jnp.where(den_r > 0, den_r, 1.0))
        return apa, apr

    def pair_body(pair, carry):
        vecA, vecR = carry
        row0 = pair * 2
        li = (pair % 8) * 2
        apa0, apr0 = row_work(row0, 0, sem0)
        vecA = jnp.where(lane == li, apa0, vecA)
        vecR = jnp.where(lane == li, apr0, vecR)

        @pl.when(row0 + 2 < rows_per_w)
        def _():
            start_dma(row0 + 2, p_buf.at[0], t_buf.at[0], sem0)

        apa1, apr1 = row_work(row0 + 1, 1, sem1)
        vecA = jnp.where(lane == li + 1, apa1, vecA)
        vecR = jnp.where(lane == li + 1, apr1, vecR)

        @pl.when(row0 + 3 < rows_per_w)
        def _():
            start_dma(row0 + 3, p_buf.at[1], t_buf.at[1], sem1)

        @pl.when(pair % 8 == 7)
        def _(_vA=vecA, _vR=vecR):
            base = (pair // 8) * _G
            outa_v[pl.ds(base, _G)] = _vA
            outr_v[pl.ds(base, _G)] = _vR

        return (vecA, vecR)

    zf16 = jnp.zeros((_G,), jnp.float32)
    lax.fori_loop(0, rows_per_w // 2, pair_body, (zf16, zf16))
    pltpu.sync_copy(outa_v, outa_hbm.at[pl.ds(g_base, rows_per_w)])
    pltpu.sync_copy(outr_v, outr_hbm.at[pl.ds(g_base, rows_per_w)])


def _sc_ap(pred, target, s_rows):
    rows_per_w = s_rows // _NW
    mesh = plsc.VectorSubcoreMesh(core_axis_name="c", subcore_axis_name="s")
    f = pl.kernel(
        _sc_body,
        out_type=(jax.ShapeDtypeStruct((s_rows,), jnp.float32),
                  jax.ShapeDtypeStruct((s_rows,), jnp.float32)),
        mesh=mesh,
        compiler_params=pltpu.CompilerParams(use_tc_tiling_on_sc=False),
        scratch_types=[
            pltpu.VMEM((2, _N), jnp.float32),
            pltpu.VMEM((2, _N), jnp.float32),
            pltpu.VMEM((_N,), jnp.float32),
            pltpu.VMEM((_N,), jnp.float32),
            pltpu.VMEM((rows_per_w,), jnp.float32),
            pltpu.VMEM((rows_per_w,), jnp.float32),
            pltpu.SemaphoreType.DMA,
            pltpu.SemaphoreType.DMA,
        ],
    )
    return f(pred, target)


_R = 256


def _tc_body(pred_ref, target_ref, out_ref, *, row0):
    rblk = pl.program_id(0)
    p = pred_ref[0]
    t = target_ref[0]
    R, N = p.shape

    row_i = (row0 + rblk * R) % _N + lax.broadcasted_iota(jnp.int32, (R, N), 0)
    col = lax.broadcasted_iota(jnp.int32, (R, N), 1)
    vt = jnp.where(col == row_i, 0.0, t)
    nt = 1.0 - t
    a = p * vt
    r = (1.0 - p) * nt

    kA1 = (jnp.sum(vt, axis=1) * 0.3).astype(jnp.int32) + 1
    kR1 = (jnp.sum(nt, axis=1) * 0.3).astype(jnp.int32) + 1

    bitsA = lax.bitcast_convert_type(a, jnp.int32)
    bitsR = lax.bitcast_convert_type(r, jnp.int32)

    def step(i, carry):
        prefA, prefR = carry
        bit = jnp.int32(1) << (29 - i)
        candA = prefA | bit
        candR = prefR | bit
        cA = jnp.sum((bitsA >= candA[:, None]).astype(jnp.int32), axis=1)
        cR = jnp.sum((bitsR >= candR[:, None]).astype(jnp.int32), axis=1)
        return (jnp.where(cA >= kA1, candA, prefA),
                jnp.where(cR >= kR1, candR, prefR))

    zero = jnp.zeros((R,), jnp.int32)
    prefA, prefR = lax.fori_loop(0, 18, step, (zero, zero))
    low = jnp.int32((1 << 12) - 1)
    thA = lax.bitcast_convert_type(prefA | low, jnp.float32)[:, None]
    thR = lax.bitcast_convert_type(prefR | low, jnp.float32)[:, None]

    mA = jnp.where(a <= thA, vt, 0.0)
    mR = jnp.where(r <= thR, nt, 0.0)
    sA = jnp.sum(a * mA, axis=1)
    cA = jnp.sum(mA, axis=1)
    sR = jnp.sum(r * mR, axis=1)
    cR = jnp.sum(mR, axis=1)
    apA = sA / jnp.where(cA > 0, cA, 1.0)
    apR = sR / jnp.where(cR > 0, cR, 1.0)
    lossA = -jnp.maximum(jnp.log(apA), -100.0)
    lossR = -jnp.maximum(jnp.log(apR), -100.0)
    blk = jnp.sum(lossA + lossR)

    @pl.when(rblk == 0)
    def _():
        out_ref[...] = jnp.zeros_like(out_ref)

    out_ref[...] += jnp.reshape(blk, (1, 1))


def _tc_partial(pred, target, row0, nrows):
    grid = (nrows // _R,)

    def imap(rb):
        flat = row0 // _R + rb
        return (flat // (_N // _R), flat % (_N // _R), 0)

    return pl.pallas_call(
        functools.partial(_tc_body, row0=row0),
        grid=grid,
        in_specs=[
            pl.BlockSpec((1, _R, _N), imap),
            pl.BlockSpec((1, _R, _N), imap),
        ],
        out_specs=pl.BlockSpec((1, 1), lambda rb: (0, 0)),
        out_shape=jax.ShapeDtypeStruct((1, 1), jnp.float32),
    )(pred, target)


def _combine_body(apa_ref, apr_ref, part_ref, out_ref, *, n_total):
    la = -jnp.maximum(jnp.log(apa_ref[...]), -100.0)
    lr = -jnp.maximum(jnp.log(apr_ref[...]), -100.0)
    tot = jnp.sum(la + lr) + part_ref[0, 0]
    out_ref[...] = jnp.reshape(tot * (1.0 / n_total), (1, 1))


def _combine(apa, apr, partial, n_total):
    s = apa.shape[0]
    out = pl.pallas_call(
        functools.partial(_combine_body, n_total=float(n_total)),
        out_shape=jax.ShapeDtypeStruct((1, 1), jnp.float32),
    )(apa.reshape(s // 128, 128), apr.reshape(s // 128, 128), partial)
    return out[0, 0]


_S_SC = 1536


def kernel(pred, target):
    B, N, _ = pred.shape
    total = B * N
    apa, apr = _sc_ap(pred, target, _S_SC)
    partial = _tc_partial(pred, target, _S_SC, total - _S_SC)
    return _combine(apa, apr, partial, total)


# speedup vs baseline: 1.1275x; 1.1275x over previous
import functools

import jax
import jax.numpy as jnp
from jax import lax
from jax.experimental import pallas as pl
from jax.experimental.pallas import tpu as pltpu
from jax.experimental.pallas import tpu_sc as plsc

_NW = 32
_G = 16
_N = 2048


def _drain(v):
    s = v[0]
    for l in range(1, _G):
        s = s + v[l]
    return s


def _sc_body(pred_hbm, target_hbm, outa_hbm, outr_hbm,
             p_buf, t_buf, a_row, r_row, outa_v, outr_v, sem0, sem1):
    wid = lax.axis_index("s") * 2 + lax.axis_index("c")
    rows_per_w = outa_v.shape[0]
    lane = lax.iota(jnp.int32, _G)
    g_base = wid * rows_per_w
    nchunk = _N // _G

    def start_dma(row, slot_p, slot_t, s):
        g = g_base + row
        cp = pltpu.make_async_copy(pred_hbm.at[g], slot_p, s)
        ct = pltpu.make_async_copy(target_hbm.at[g], slot_t, s)
        cp.start()
        ct.start()
        return cp, ct

    start_dma(0, p_buf.at[0], t_buf.at[0], sem0)
    start_dma(1, p_buf.at[1], t_buf.at[1], sem1)

    def row_work(row, slot, s):
        g = g_base + row
        pltpu.make_async_copy(pred_hbm.at[g], p_buf.at[slot], s).wait()
        pltpu.make_async_copy(target_hbm.at[g], t_buf.at[slot], s).wait()
        i_col = (g_base + row) % _N
        p_r = p_buf.at[slot]
        t_r = t_buf.at[slot]

        def prep_c(c, acc):
            cA, cR = acc
            col = c * _G + lane
            p16 = p_r[pl.ds(c * _G, _G)]
            t16 = t_r[pl.ds(c * _G, _G)]
            vt = jnp.where(col == i_col, 0.0, t16)
            nt = 1.0 - t16
            a_row[pl.ds(c * _G, _G)] = p16 * vt
            r_row[pl.ds(c * _G, _G)] = (1.0 - p16) * nt
            return (cA + vt, cR + nt)

        zf = jnp.zeros((_G,), jnp.float32)
        cAv, cRv = lax.fori_loop(0, nchunk, prep_c, (zf, zf), unroll=4)
        kA1 = (_drain(cAv) * 0.3).astype(jnp.int32) + 1
        kR1 = (_drain(cRv) * 0.3).astype(jnp.int32) + 1

        loA = jnp.float32(0.0)
        hiA = jnp.float32(1.001)
        loR = jnp.float32(0.0)
        hiR = jnp.float32(1.001)
        zi = jnp.zeros((_G,), jnp.int32)
        one = jnp.ones((_G,), jnp.int32)
        for _ in range(18):
            midA = (loA + hiA) * 0.5
            midR = (loR + hiR) * 0.5

            def cnt_c(c, acc, _mA=midA, _mR=midR):
                aA, aR = acc
                a16 = a_row[pl.ds(c * _G, _G)]
                r16 = r_row[pl.ds(c * _G, _G)]
                aA = aA + jnp.where(a16 >= _mA, one, zi)
                aR = aR + jnp.where(r16 >= _mR, one, zi)
                return (aA, aR)

            accA, accR = lax.fori_loop(0, nchunk, cnt_c, (zi, zi), unroll=8)
            okA = _drain(accA) >= kA1
            okR = _drain(accR) >= kR1
            loA = jnp.where(okA, midA, loA)
            hiA = jnp.where(okA, hiA, midA)
            loR = jnp.where(okR, midR, loR)
            hiR = jnp.where(okR, hiR, midR)

        def sum_c(c, acc):
            sA, mAc, sR, mRc = acc
            col = c * _G + lane
            a16 = a_row[pl.ds(c * _G, _G)]
            r16 = r_row[pl.ds(c * _G, _G)]
            t16 = t_r[pl.ds(c * _G, _G)]
            vt = jnp.where(col == i_col, 0.0, t16)
            nt = 1.0 - t16
            mA = a16 <= hiA
            mR = r16 <= hiR
            return (sA + jnp.where(mA, a16, 0.0),
                    mAc + jnp.where(mA, vt, 0.0),
                    sR + jnp.where(mR, r16, 0.0),
                    mRc + jnp.where(mR, nt, 0.0))

        sA, mAc, sR, mRc = lax.fori_loop(
            0, nchunk, sum_c, (zf, zf, zf, zf), unroll=4)
        den_a = _drain(mAc)
        den_r = _drain(mRc)
        zv = jnp.zeros((_G,), jnp.float32)
        apa = (zv + _drain(sA)) / (zv + jnp.where(den_a > 0, den_a, 1.0))
        apr = (zv + _drain(sR)) / (zv + jnp.where(den_r > 0, den_r, 1.0))
        return apa, apr

    def pair_body(pair, carry):
        vecA, vecR = carry
        row0 = pair * 2
        li = (pair % 8) * 2
        apa0, apr0 = row_work(row0, 0, sem0)
        vecA = jnp.where(lane == li, apa0, vecA)
        vecR = jnp.where(lane == li, apr0, vecR)

        @pl.when(row0 + 2 < rows_per_w)
        def _():
            start_dma(row0 + 2, p_buf.at[0], t_buf.at[0], sem0)

        apa1, apr1 = row_work(row0 + 1, 1, sem1)
        vecA = jnp.where(lane == li + 1, apa1, vecA)
        vecR = jnp.where(lane == li + 1, apr1, vecR)

        @pl.when(row0 + 3 < rows_per_w)
        def _():
            start_dma(row0 + 3, p_buf.at[1], t_buf.at[1], sem1)

        @pl.when(pair % 8 == 7)
        def _(_vA=vecA, _vR=vecR):
            base = (pair // 8) * _G
            outa_v[pl.ds(base, _G)] = _vA
            outr_v[pl.ds(base, _G)] = _vR

        return (vecA, vecR)

    zf16 = jnp.zeros((_G,), jnp.float32)
    lax.fori_loop(0, rows_per_w // 2, pair_body, (zf16, zf16))
    pltpu.sync_copy(outa_v, outa_hbm.at[pl.ds(g_base, rows_per_w)])
    pltpu.sync_copy(outr_v, outr_hbm.at[pl.ds(g_base, rows_per_w)])


def _sc_ap(pred, target, s_rows):
    rows_per_w = s_rows // _NW
    mesh = plsc.VectorSubcoreMesh(core_axis_name="c", subcore_axis_name="s")
    f = pl.kernel(
        _sc_body,
        out_type=(jax.ShapeDtypeStruct((s_rows,), jnp.float32),
                  jax.ShapeDtypeStruct((s_rows,), jnp.float32)),
        mesh=mesh,
        compiler_params=pltpu.CompilerParams(use_tc_tiling_on_sc=False),
        scratch_types=[
            pltpu.VMEM((2, _N), jnp.float32),
            pltpu.VMEM((2, _N), jnp.float32),
            pltpu.VMEM((_N,), jnp.float32),
            pltpu.VMEM((_N,), jnp.float32),
            pltpu.VMEM((rows_per_w,), jnp.float32),
            pltpu.VMEM((rows_per_w,), jnp.float32),
            pltpu.SemaphoreType.DMA,
            pltpu.SemaphoreType.DMA,
        ],
    )
    return f(pred, target)


_R = 256


def _tc_body(pred_ref, target_ref, out_ref, *, row0):
    rblk = pl.program_id(0)
    p = pred_ref[0]
    t = target_ref[0]
    R, N = p.shape

    row_i = (row0 + rblk * R) % _N + lax.broadcasted_iota(jnp.int32, (R, N), 0)
    col = lax.broadcasted_iota(jnp.int32, (R, N), 1)
    vt = jnp.where(col == row_i, 0.0, t)
    nt = 1.0 - t
    a = p * vt
    r = (1.0 - p) * nt

    kA1 = (jnp.sum(vt, axis=1) * 0.3).astype(jnp.int32) + 1
    kR1 = (jnp.sum(nt, axis=1) * 0.3).astype(jnp.int32) + 1

    bitsA = lax.bitcast_convert_type(a, jnp.int32)
    bitsR = lax.bitcast_convert_type(r, jnp.int32)

    def step(i, carry):
        prefA, prefR = carry
        bit = jnp.int32(1) << (29 - i)
        candA = prefA | bit
        candR = prefR | bit
        cA = jnp.sum((bitsA >= candA[:, None]).astype(jnp.int32), axis=1)
        cR = jnp.sum((bitsR >= candR[:, None]).astype(jnp.int32), axis=1)
        return (jnp.where(cA >= kA1, candA, prefA),
                jnp.where(cR >= kR1, candR, prefR))

    zero = jnp.zeros((R,), jnp.int32)
    prefA, prefR = lax.fori_loop(0, 18, step, (zero, zero))
    low = jnp.int32((1 << 12) - 1)
    thA = lax.bitcast_convert_type(prefA | low, jnp.float32)[:, None]
    thR = lax.bitcast_convert_type(prefR | low, jnp.float32)[:, None]

    mA = jnp.where(a <= thA, vt, 0.0)
    mR = jnp.where(r <= thR, nt, 0.0)
    sA = jnp.sum(a * mA, axis=1)
    cA = jnp.sum(mA, axis=1)
    sR = jnp.sum(r * mR, axis=1)
    cR = jnp.sum(mR, axis=1)
    apA = sA / jnp.where(cA > 0, cA, 1.0)
    apR = sR / jnp.where(cR > 0, cR, 1.0)
    lossA = -jnp.maximum(jnp.log(apA), -100.0)
    lossR = -jnp.maximum(jnp.log(apR), -100.0)
    blk = jnp.sum(lossA + lossR)

    @pl.when(rblk == 0)
    def _():
        out_ref[...] = jnp.zeros_like(out_ref)

    out_ref[...] += jnp.reshape(blk, (1, 1))


def _tc_partial(pred, target, row0, nrows):
    grid = (nrows // _R,)

    def imap(rb):
        flat = row0 // _R + rb
        return (flat // (_N // _R), flat % (_N // _R), 0)

    return pl.pallas_call(
        functools.partial(_tc_body, row0=row0),
        grid=grid,
        in_specs=[
            pl.BlockSpec((1, _R, _N), imap),
            pl.BlockSpec((1, _R, _N), imap),
        ],
        out_specs=pl.BlockSpec((1, 1), lambda rb: (0, 0)),
        out_shape=jax.ShapeDtypeStruct((1, 1), jnp.float32),
    )(pred, target)


def _combine_body(apa_ref, apr_ref, part_ref, out_ref, *, n_total):
    la = -jnp.maximum(jnp.log(apa_ref[...]), -100.0)
    lr = -jnp.maximum(jnp.log(apr_ref[...]), -100.0)
    tot = jnp.sum(la + lr) + part_ref[0, 0]
    out_ref[...] = jnp.reshape(tot * (1.0 / n_total), (1, 1))


def _combine(apa, apr, partial, n_total):
    s = apa.shape[0]
    out = pl.pallas_call(
        functools.partial(_combine_body, n_total=float(n_total)),
        out_shape=jax.ShapeDtypeStruct((1, 1), jnp.float32),
    )(apa.reshape(s // 128, 128), apr.reshape(s // 128, 128), partial)
    return out[0, 0]


_S_SC = 2048


def kernel(pred, target):
    B, N, _ = pred.shape
    total = B * N
    apa, apr = _sc_ap(pred[0], target[0], _S_SC)
    partial = _tc_partial(pred, target, _S_SC, total - _S_SC)
    return _combine(apa, apr, partial, total)


# speedup vs baseline: 1.1297x; 1.0019x over previous
import functools

import jax
import jax.numpy as jnp
from jax import lax
from jax.experimental import pallas as pl
from jax.experimental.pallas import tpu as pltpu
from jax.experimental.pallas import tpu_sc as plsc

_NW = 32
_G = 16
_N = 2048


def _drain(v):
    s = v[0]
    for l in range(1, _G):
        s = s + v[l]
    return s


def _sc_body(pred_hbm, target_hbm, outa_hbm, outr_hbm,
             p_buf, t_buf, a_row, r_row, outa_v, outr_v, sem0, sem1):
    wid = lax.axis_index("s") * 2 + lax.axis_index("c")
    rows_per_w = outa_v.shape[0]
    lane = lax.iota(jnp.int32, _G)
    g_base = wid * rows_per_w
    nchunk = _N // _G

    def start_dma(row, slot_p, slot_t, s):
        g = g_base + row
        cp = pltpu.make_async_copy(pred_hbm.at[g], slot_p, s)
        ct = pltpu.make_async_copy(target_hbm.at[g], slot_t, s)
        cp.start()
        ct.start()
        return cp, ct

    start_dma(0, p_buf.at[0], t_buf.at[0], sem0)
    start_dma(1, p_buf.at[1], t_buf.at[1], sem1)

    def row_work(row, slot, s):
        g = g_base + row
        pltpu.make_async_copy(pred_hbm.at[g], p_buf.at[slot], s).wait()
        pltpu.make_async_copy(target_hbm.at[g], t_buf.at[slot], s).wait()
        i_col = (g_base + row) % _N
        p_r = p_buf.at[slot]
        t_r = t_buf.at[slot]

        def prep_c(c, acc):
            cA, cR = acc
            col = c * _G + lane
            p16 = p_r[pl.ds(c * _G, _G)]
            t16 = t_r[pl.ds(c * _G, _G)]
            vt = jnp.where(col == i_col, 0.0, t16)
            nt = 1.0 - t16
            a_row[pl.ds(c * _G, _G)] = p16 * vt
            r_row[pl.ds(c * _G, _G)] = (1.0 - p16) * nt
            return (cA + vt, cR + nt)

        zf = jnp.zeros((_G,), jnp.float32)
        cAv, cRv = lax.fori_loop(0, nchunk, prep_c, (zf, zf), unroll=4)
        kA1 = (_drain(cAv) * 0.3).astype(jnp.int32) + 1
        kR1 = (_drain(cRv) * 0.3).astype(jnp.int32) + 1

        loA = jnp.float32(0.0)
        hiA = jnp.float32(1.001)
        loR = jnp.float32(0.0)
        hiR = jnp.float32(1.001)
        zi = jnp.zeros((_G,), jnp.int32)
        one = jnp.ones((_G,), jnp.int32)
        for _ in range(18):
            midA = (loA + hiA) * 0.5
            midR = (loR + hiR) * 0.5

            def cnt_c(c, acc, _mA=midA, _mR=midR):
                aA, aR = acc
                a16 = a_row[pl.ds(c * _G, _G)]
                r16 = r_row[pl.ds(c * _G, _G)]
                aA = aA + jnp.where(a16 >= _mA, one, zi)
                aR = aR + jnp.where(r16 >= _mR, one, zi)
                return (aA, aR)

            accA, accR = lax.fori_loop(0, nchunk, cnt_c, (zi, zi), unroll=8)
            okA = _drain(accA) >= kA1
            okR = _drain(accR) >= kR1
            loA = jnp.where(okA, midA, loA)
            hiA = jnp.where(okA, hiA, midA)
            loR = jnp.where(okR, midR, loR)
            hiR = jnp.where(okR, hiR, midR)

        def sum_c(c, acc):
            sA, mAc, sR, mRc = acc
            col = c * _G + lane
            a16 = a_row[pl.ds(c * _G, _G)]
            r16 = r_row[pl.ds(c * _G, _G)]
            t16 = t_r[pl.ds(c * _G, _G)]
            vt = jnp.where(col == i_col, 0.0, t16)
            nt = 1.0 - t16
            mA = a16 <= hiA
            mR = r16 <= hiR
            return (sA + jnp.where(mA, a16, 0.0),
                    mAc + jnp.where(mA, vt, 0.0),
                    sR + jnp.where(mR, r16, 0.0),
                    mRc + jnp.where(mR, nt, 0.0))

        sA, mAc, sR, mRc = lax.fori_loop(
            0, nchunk, sum_c, (zf, zf, zf, zf), unroll=4)
        den_a = _drain(mAc)
        den_r = _drain(mRc)
        zv = jnp.zeros((_G,), jnp.float32)
        apa = (zv + _drain(sA)) / (zv + jnp.where(den_a > 0, den_a, 1.0))
        apr = (zv + _drain(sR)) / (zv + jnp.where(den_r > 0, den_r, 1.0))
        return apa, apr

    def pair_body(pair, carry):
        vecA, vecR = carry
        row0 = pair * 2
        li = (pair % 8) * 2
        apa0, apr0 = row_work(row0, 0, sem0)
        vecA = jnp.where(lane == li, apa0, vecA)
        vecR = jnp.where(lane == li, apr0, vecR)

        @pl.when(row0 + 2 < rows_per_w)
        def _():
            start_dma(row0 + 2, p_buf.at[0], t_buf.at[0], sem0)

        apa1, apr1 = row_work(row0 + 1, 1, sem1)
        vecA = jnp.where(lane == li + 1, apa1, vecA)
        vecR = jnp.where(lane == li + 1, apr1, vecR)

        @pl.when(row0 + 3 < rows_per_w)
        def _():
            start_dma(row0 + 3, p_buf.at[1], t_buf.at[1], sem1)

        @pl.when(pair % 8 == 7)
        def _(_vA=vecA, _vR=vecR):
            base = (pair // 8) * _G
            outa_v[pl.ds(base, _G)] = _vA
            outr_v[pl.ds(base, _G)] = _vR

        return (vecA, vecR)

    zf16 = jnp.zeros((_G,), jnp.float32)
    lax.fori_loop(0, rows_per_w // 2, pair_body, (zf16, zf16))
    pltpu.sync_copy(outa_v, outa_hbm.at[pl.ds(g_base, rows_per_w)])
    pltpu.sync_copy(outr_v, outr_hbm.at[pl.ds(g_base, rows_per_w)])


def _sc_ap(pred, target, s_rows):
    rows_per_w = s_rows // _NW
    mesh = plsc.VectorSubcoreMesh(core_axis_name="c", subcore_axis_name="s")
    f = pl.kernel(
        _sc_body,
        out_type=(jax.ShapeDtypeStruct((s_rows,), jnp.float32),
                  jax.ShapeDtypeStruct((s_rows,), jnp.float32)),
        mesh=mesh,
        compiler_params=pltpu.CompilerParams(use_tc_tiling_on_sc=False),
        scratch_types=[
            pltpu.VMEM((2, _N), jnp.float32),
            pltpu.VMEM((2, _N), jnp.float32),
            pltpu.VMEM((_N,), jnp.float32),
            pltpu.VMEM((_N,), jnp.float32),
            pltpu.VMEM((rows_per_w,), jnp.float32),
            pltpu.VMEM((rows_per_w,), jnp.float32),
            pltpu.SemaphoreType.DMA,
            pltpu.SemaphoreType.DMA,
        ],
    )
    return f(pred, target)


_R = 256


def _tc_body(pred_ref, target_ref, out_ref, *, row0):
    rblk = pl.program_id(0)
    p = pred_ref[0]
    t = target_ref[0]
    R, N = p.shape

    row_i = (row0 + rblk * R) % _N + lax.broadcasted_iota(jnp.int32, (R, N), 0)
    col = lax.broadcasted_iota(jnp.int32, (R, N), 1)
    vt = jnp.where(col == row_i, 0.0, t)
    nt = 1.0 - t
    a = p * vt
    r = (1.0 - p) * nt

    kA1 = (jnp.sum(vt, axis=1) * 0.3).astype(jnp.int32) + 1
    kR1 = (jnp.sum(nt, axis=1) * 0.3).astype(jnp.int32) + 1

    bitsA = lax.bitcast_convert_type(a, jnp.int32)
    bitsR = lax.bitcast_convert_type(r, jnp.int32)

    def step(i, carry):
        prefA, prefR = carry
        bit = jnp.int32(1) << (29 - i)
        candA = prefA | bit
        candR = prefR | bit
        cA = jnp.sum((bitsA >= candA[:, None]).astype(jnp.int32), axis=1)
        cR = jnp.sum((bitsR >= candR[:, None]).astype(jnp.int32), axis=1)
        return (jnp.where(cA >= kA1, candA, prefA),
                jnp.where(cR >= kR1, candR, prefR))

    zero = jnp.zeros((R,), jnp.int32)
    prefA, prefR = lax.fori_loop(0, 18, step, (zero, zero))
    low = jnp.int32((1 << 12) - 1)
    thA = lax.bitcast_convert_type(prefA | low, jnp.float32)[:, None]
    thR = lax.bitcast_convert_type(prefR | low, jnp.float32)[:, None]

    mA = jnp.where(a <= thA, vt, 0.0)
    mR = jnp.where(r <= thR, nt, 0.0)
    sA = jnp.sum(a * mA, axis=1)
    cA = jnp.sum(mA, axis=1)
    sR = jnp.sum(r * mR, axis=1)
    cR = jnp.sum(mR, axis=1)
    apA = sA / jnp.where(cA > 0, cA, 1.0)
    apR = sR / jnp.where(cR > 0, cR, 1.0)
    lossA = -jnp.maximum(jnp.log(apA), -100.0)
    lossR = -jnp.maximum(jnp.log(apR), -100.0)
    blk = jnp.sum(lossA + lossR)

    @pl.when(rblk == 0)
    def _():
        out_ref[...] = jnp.zeros_like(out_ref)

    out_ref[...] += jnp.reshape(blk, (1, 1))


def _tc_partial(pred, target, row0, nrows):
    grid = (nrows // _R,)

    def imap(rb):
        flat = row0 // _R + rb
        return (flat // (_N // _R), flat % (_N // _R), 0)

    return pl.pallas_call(
        functools.partial(_tc_body, row0=row0),
        grid=grid,
        in_specs=[
            pl.BlockSpec((1, _R, _N), imap),
            pl.BlockSpec((1, _R, _N), imap),
        ],
        out_specs=pl.BlockSpec((1, 1), lambda rb: (0, 0)),
        out_shape=jax.ShapeDtypeStruct((1, 1), jnp.float32),
    )(pred, target)


def _combine_body(apa_ref, apr_ref, part_ref, out_ref, *, n_total):
    la = -jnp.maximum(jnp.log(apa_ref[...]), -100.0)
    lr = -jnp.maximum(jnp.log(apr_ref[...]), -100.0)
    tot = jnp.sum(la + lr) + part_ref[0, 0]
    out_ref[...] = jnp.reshape(tot * (1.0 / n_total), (1, 1))


def _combine(apa, apr, partial, n_total):
    s = apa.shape[0]
    out = pl.pallas_call(
        functools.partial(_combine_body, n_total=float(n_total)),
        out_shape=jax.ShapeDtypeStruct((1, 1), jnp.float32),
    )(apa.reshape(s // 128, 128), apr.reshape(s // 128, 128), partial)
    return out[0, 0]


_S_SC = 1536


def kernel(pred, target):
    B, N, _ = pred.shape
    total = B * N
    apa, apr = _sc_ap(pred[0], target[0], _S_SC)
    partial = _tc_partial(pred, target, _S_SC, total - _S_SC)
    return _combine(apa, apr, partial, total)
